# baseline (speedup 1.0000x reference)
import functools

import jax
import jax.numpy as jnp
from jax import lax
from jax.experimental import pallas as pl
from jax.experimental.pallas import tpu as pltpu
from jax.experimental.pallas import tpu_sc as plsc

D = 128
NCORES = 2
NSUB = 16
NW = NCORES * NSUB
BLK = 40
NPAD = 10240
ZROWS = 160

def _mm(a, b):
    return jax.lax.dot_general(a, b, (((1,), (0,)), ((), ())),
                               preferred_element_type=jnp.float32)


def _pad2(w, r, c):
    return jnp.pad(w, ((0, r - w.shape[0]), (0, c - w.shape[1])))


def _pad1(b, n, fill=0.0):
    return jnp.pad(b, (0, n - b.shape[0]), constant_values=fill).reshape(1, n)


def _proj(x, w_p, b_p, rb):
    n, k = x.shape

    def body(x_ref, w_ref, b_ref, o_ref):
        o_ref[...] = _mm(x_ref[...], w_ref[...]) + b_ref[...]

    return pl.pallas_call(
        body,
        grid=(n // rb,),
        in_specs=[pl.BlockSpec((rb, k), lambda i: (i, 0)),
                  pl.BlockSpec((k, D), lambda i: (0, 0)),
                  pl.BlockSpec((1, D), lambda i: (0, 0))],
        out_specs=pl.BlockSpec((rb, D), lambda i: (i, 0)),
        out_shape=jax.ShapeDtypeStruct((n, D), jnp.float32),
    )(x, w_p, b_p)


def _layer_update(h_pad, parts, w1, b1, w2, b2, g, bt, m, v, head=None):
    n = h_pad.shape[0]
    rb = 2000

    def update(h, agg, w1r, b1r, w2r, b2r, gr, btr, mr, vr):
        z = h + agg
        z = _mm(jnp.maximum(_mm(z, w1r) + b1r, 0.0), w2r) + b2r
        z = (z - mr) * jax.lax.rsqrt(vr + 1e-5) * gr + btr
        return (h + jnp.maximum(z, 0.0)) * 0.5

    if head is None:
        def body(h_ref, p_ref, w1r, b1r, w2r, b2r, gr, btr, mr, vr,
                 o_ref):
            o_ref[...] = update(h_ref[...], p_ref[0] + p_ref[1],
                                w1r[...], b1r[...], w2r[...], b2r[...],
                                gr[...], btr[...], mr[...], vr[...])
        extra_in, extra_specs = [], []
    else:
        mw1, mb1, mw2, mb2, mw3, mb3 = head

        def body(h_ref, p_ref, w1r, b1r, w2r, b2r, gr, btr, mr, vr,
                 mw1r, mb1r, mw2r, mb2r, mw3r, mb3r, o_ref):
            hn = update(h_ref[...], p_ref[0] + p_ref[1],
                        w1r[...], b1r[...], w2r[...], b2r[...],
                        gr[...], btr[...], mr[...], vr[...])
            o1 = jnp.maximum(_mm(hn, mw1r[...]) + mb1r[...], 0.0)
            o2 = jnp.maximum(_mm(o1, mw2r[...]) + mb2r[...], 0.0)
            o_ref[...] = _mm(o2, mw3r[...]) + mb3r[...]
        extra_in = [mw1, mb1, mw2, mb2, mw3, mb3]
        extra_specs = [
            pl.BlockSpec((D, 64), lambda i: (0, 0)),
            pl.BlockSpec((1, 64), lambda i: (0, 0)),
            pl.BlockSpec((64, 32), lambda i: (0, 0)),
            pl.BlockSpec((1, 32), lambda i: (0, 0)),
            pl.BlockSpec((32, D), lambda i: (0, 0)),
            pl.BlockSpec((1, D), lambda i: (0, 0)),
        ]

    return pl.pallas_call(
        body,
        grid=(n // rb,),
        in_specs=[pl.BlockSpec((rb, D), lambda i: (i, 0)),
                  pl.BlockSpec((NCORES, rb, D), lambda i: (0, i, 0)),
                  pl.BlockSpec((D, D), lambda i: (0, 0)),
                  pl.BlockSpec((1, D), lambda i: (0, 0)),
                  pl.BlockSpec((D, D), lambda i: (0, 0)),
                  pl.BlockSpec((1, D), lambda i: (0, 0)),
                  pl.BlockSpec((1, D), lambda i: (0, 0)),
                  pl.BlockSpec((1, D), lambda i: (0, 0)),
                  pl.BlockSpec((1, D), lambda i: (0, 0)),
                  pl.BlockSpec((1, D), lambda i: (0, 0))] + extra_specs,
        out_specs=pl.BlockSpec((rb, D), lambda i: (i, 0)),
        out_shape=jax.ShapeDtypeStruct((n, D), jnp.float32),
    )(h_pad, parts, w1, b1, w2, b2, g, bt, m, v, *extra_in)


def _sc_aggregate(h_pad, packed2, ea_pad):
    epw = ea_pad.shape[0] // NW
    nblk = epw // BLK
    epw_pad = packed2.shape[1]
    rps = NPAD // NSUB
    mesh = plsc.VectorSubcoreMesh(core_axis_name="c", subcore_axis_name="s")
    offs = tuple(sorted(set(list(range(0, BLK - 15, 16)) + [BLK - 16])))

    @functools.partial(
        pl.kernel,
        out_type=jax.ShapeDtypeStruct((NCORES, NPAD, D), jnp.float32),
        mesh=mesh,
        scratch_types=[
            pltpu.VMEM((epw_pad,), jnp.int32),
            pltpu.VMEM((2, BLK), jnp.int32),
            pltpu.VMEM((2, BLK), jnp.int32),
            pltpu.VMEM((2, BLK, D), jnp.float32),
            pltpu.VMEM((2, BLK, D), jnp.float32),
            pltpu.VMEM((2, BLK, D), jnp.float32),
            pltpu.VMEM_SHARED((NPAD, D), jnp.float32),
            pltpu.SemaphoreType.DMA,
            pltpu.SemaphoreType.DMA,
            pltpu.SemaphoreType.DMA,
            pltpu.SemaphoreType.DMA,
            pltpu.SemaphoreType.DMA,
            pltpu.SemaphoreType.DMA,
            pltpu.SemaphoreType.DMA,
            pltpu.SemaphoreType.DMA,
        ],
    )
    def k(h_hbm, pk_hbm, ea_hbm, out_hbm,
          pk_all, sidx, didx, gat, eab, msg, agg_sp,
          isem, zsem, gsem0, gsem1, esem0, esem1, ssem0, ssem1):
        core = lax.axis_index("c")
        sub = lax.axis_index("s")
        wid = sub * NCORES + core
        ebase = wid * epw
        gsem = (gsem0, gsem1)
        esem = (esem0, esem1)
        ssem = (ssem0, ssem1)

        pltpu.async_copy(pk_hbm.at[wid], pk_all, isem)

        for p in range(2):
            @pl.loop(0, BLK)
            def _(r):
                for j in range(D // 16):
                    msg.at[p, r, pl.ds(j * 16, 16)][...] = jnp.zeros(
                        (16,), jnp.float32)

        @pl.loop(0, rps // BLK)
        def _(t):
            pltpu.async_copy(
                msg.at[0], agg_sp.at[pl.ds(sub * rps + t * BLK, BLK)], zsem)
        pltpu.make_async_copy(pk_hbm.at[wid], pk_all, isem).wait()

        @pl.loop(0, rps // BLK)
        def _(t):
            pltpu.make_async_copy(
                msg.at[0], agg_sp.at[pl.ds(sub * rps + t * BLK, BLK)],
                zsem).wait()

        def unpack_src(b, p):
            for off in offs:
                sidx.at[p, pl.ds(off, 16)][...] = jnp.bitwise_and(
                    pk_all.at[pl.ds(b * BLK + off, 16)][...], 16383)

        def unpack_dst(b, p):
            for off in offs:
                didx.at[p, pl.ds(off, 16)][...] = jax.lax.shift_right_logical(
                    pk_all.at[pl.ds(b * BLK + off, 16)][...], 14)

        def fetch(b, p):
            pltpu.async_copy(ea_hbm.at[pl.ds(ebase + b * BLK, BLK)],
                             eab.at[p], esem[p])
            pltpu.async_copy(h_hbm.at[sidx.at[p]], gat.at[p], gsem[p])

        def wait_fetch(b, p):
            pltpu.make_async_copy(ea_hbm.at[pl.ds(ebase + b * BLK, BLK)],
                                  eab.at[p], esem[p]).wait()
            pltpu.make_async_copy(h_hbm.at[sidx.at[p]], gat.at[p],
                                  gsem[p]).wait()

        def wait_scat(b, p):
            pltpu.make_async_copy(msg.at[p], agg_sp.at[didx.at[p]],
                                  ssem[p]).wait()

        def compute(p):
            @pl.loop(0, BLK, unroll=2)
            def _(r):
                for j in range(D // 16 - 1):
                    sl = pl.ds(j * 16, 16)
                    msg.at[p, r, sl][...] = jnp.maximum(
                        gat.at[p, r, sl][...] + eab.at[p, r, sl][...], 0.0)

        def scat(p):
            pltpu.async_copy(msg.at[p], agg_sp.at[didx.at[p]],
                             ssem[p], add=True)

        unpack_src(0, 0)
        unpack_src(1, 1)
        plsc.subcore_barrier()
        fetch(0, 0)
        fetch(1, 1)

        @pl.loop(0, nblk // 2)
        def _(t):
            for p in range(2):
                b = 2 * t + p
                wait_fetch(b, p)

                @pl.when(t > 0)
                def _():
                    wait_scat(b - 2, p)

                unpack_dst(b, p)
                compute(p)
                scat(p)

                @pl.when(b + 2 < nblk)
                def _():
                    unpack_src(b + 2, p)
                    fetch(b + 2, p)

        if nblk % 2:
            b = nblk - 1
            wait_fetch(b, 0)
            wait_scat(b - 2, 0)
            unpack_dst(b, 0)
            compute(0)
            scat(0)
            wait_scat(b - 1, 1)
            wait_scat(b, 0)
        else:
            wait_scat(nblk - 2, 0)
            wait_scat(nblk - 1, 1)
        plsc.subcore_barrier()

        @pl.loop(0, rps // ZROWS)
        def _(t):
            r0 = sub * rps + t * ZROWS
            pltpu.async_copy(agg_sp.at[pl.ds(r0, ZROWS)],
                             out_hbm.at[core, pl.ds(r0, ZROWS)], isem)

        @pl.loop(0, rps // ZROWS)
        def _(t):
            r0 = sub * rps + t * ZROWS
            pltpu.make_async_copy(agg_sp.at[pl.ds(r0, ZROWS)],
                                  out_hbm.at[core, pl.ds(r0, ZROWS)],
                                  isem).wait()

    return k(h_pad, packed2, ea_pad)


def kernel(x, edge_index, edge_attr, Wn, bn_, We, be, conv_W1, conv_b1,
           conv_W2, conv_b2, bn_gamma, bn_beta, bn_mean, bn_var,
           mW1, mb1, mW2, mb2, mW3, mb3):
    h = _proj(x, _pad2(Wn, Wn.shape[0], D), _pad1(bn_, D), 2000)
    ea = _proj(edge_attr, _pad2(We, edge_attr.shape[1], D), _pad1(be, D), 4000)
    e = edge_index.shape[1]
    epw = e // NW
    epw_pad = -(-epw // D) * D
    packed = jnp.pad((edge_index[0] + edge_index[1] * 16384).reshape(NW, epw),
                     ((0, 0), (0, epw_pad - epw)))

    head = (_pad2(mW1, D, 64), _pad1(mb1, 64),
            _pad2(mW2, 64, 32), _pad1(mb2, 32),
            _pad2(mW3, 32, D), _pad1(mb3, D))

    out_pad = None
    for i in range(conv_W1.shape[0]):
        parts = _sc_aggregate(h, packed, ea)
        args = (h, parts,
                _pad2(conv_W1[i], D, D), _pad1(conv_b1[i], D),
                _pad2(conv_W2[i], D, D), _pad1(conv_b2[i], D),
                _pad1(bn_gamma[i], D), _pad1(bn_beta[i], D),
                _pad1(bn_mean[i], D), _pad1(bn_var[i], D, fill=1.0))
        if i + 1 < conv_W1.shape[0]:
            h = _layer_update(*args)
        else:
            out_pad = _layer_update(*args, head=head)
    return out_pad[:, :mW3.shape[1]]


# speedup vs baseline: 1.0085x; 1.0085x over previous
import functools

import jax
import jax.numpy as jnp
from jax import lax
from jax.experimental import pallas as pl
from jax.experimental.pallas import tpu as pltpu
from jax.experimental.pallas import tpu_sc as plsc

D = 128
NCORES = 2
NSUB = 16
NW = NCORES * NSUB
BLK = 40
NPAD = 10240
ZROWS = 160

def _mm(a, b):
    return jax.lax.dot_general(a, b, (((1,), (0,)), ((), ())),
                               preferred_element_type=jnp.float32)


def _pad2(w, r, c):
    return jnp.pad(w, ((0, r - w.shape[0]), (0, c - w.shape[1])))


def _pad1(b, n, fill=0.0):
    return jnp.pad(b, (0, n - b.shape[0]), constant_values=fill).reshape(1, n)


def _proj(x, w_p, b_p, rb):
    n, k = x.shape

    def body(x_ref, w_ref, b_ref, o_ref):
        o_ref[...] = _mm(x_ref[...], w_ref[...]) + b_ref[...]

    return pl.pallas_call(
        body,
        grid=(n // rb,),
        in_specs=[pl.BlockSpec((rb, k), lambda i: (i, 0)),
                  pl.BlockSpec((k, D), lambda i: (0, 0)),
                  pl.BlockSpec((1, D), lambda i: (0, 0))],
        out_specs=pl.BlockSpec((rb, D), lambda i: (i, 0)),
        out_shape=jax.ShapeDtypeStruct((n, D), jnp.float32),
    )(x, w_p, b_p)


def _layer_update(h_pad, parts, w1, b1, w2, b2, g, bt, m, v, head=None):
    n = h_pad.shape[0]
    rb = 2000

    def update(h, agg, w1r, b1r, w2r, b2r, gr, btr, mr, vr):
        z = h + agg
        z = _mm(jnp.maximum(_mm(z, w1r) + b1r, 0.0), w2r) + b2r
        z = (z - mr) * jax.lax.rsqrt(vr + 1e-5) * gr + btr
        return (h + jnp.maximum(z, 0.0)) * 0.5

    if head is None:
        def body(h_ref, p_ref, w1r, b1r, w2r, b2r, gr, btr, mr, vr,
                 o_ref):
            o_ref[...] = update(h_ref[...], p_ref[0] + p_ref[1],
                                w1r[...], b1r[...], w2r[...], b2r[...],
                                gr[...], btr[...], mr[...], vr[...])
        extra_in, extra_specs = [], []
    else:
        mw1, mb1, mw2, mb2, mw3, mb3 = head

        def body(h_ref, p_ref, w1r, b1r, w2r, b2r, gr, btr, mr, vr,
                 mw1r, mb1r, mw2r, mb2r, mw3r, mb3r, o_ref):
            hn = update(h_ref[...], p_ref[0] + p_ref[1],
                        w1r[...], b1r[...], w2r[...], b2r[...],
                        gr[...], btr[...], mr[...], vr[...])
            o1 = jnp.maximum(_mm(hn, mw1r[...]) + mb1r[...], 0.0)
            o2 = jnp.maximum(_mm(o1, mw2r[...]) + mb2r[...], 0.0)
            o_ref[...] = _mm(o2, mw3r[...]) + mb3r[...]
        extra_in = [mw1, mb1, mw2, mb2, mw3, mb3]
        extra_specs = [
            pl.BlockSpec((D, 64), lambda i: (0, 0)),
            pl.BlockSpec((1, 64), lambda i: (0, 0)),
            pl.BlockSpec((64, 32), lambda i: (0, 0)),
            pl.BlockSpec((1, 32), lambda i: (0, 0)),
            pl.BlockSpec((32, D), lambda i: (0, 0)),
            pl.BlockSpec((1, D), lambda i: (0, 0)),
        ]

    return pl.pallas_call(
        body,
        grid=(n // rb,),
        in_specs=[pl.BlockSpec((rb, D), lambda i: (i, 0)),
                  pl.BlockSpec((NCORES, rb, D), lambda i: (0, i, 0)),
                  pl.BlockSpec((D, D), lambda i: (0, 0)),
                  pl.BlockSpec((1, D), lambda i: (0, 0)),
                  pl.BlockSpec((D, D), lambda i: (0, 0)),
                  pl.BlockSpec((1, D), lambda i: (0, 0)),
                  pl.BlockSpec((1, D), lambda i: (0, 0)),
                  pl.BlockSpec((1, D), lambda i: (0, 0)),
                  pl.BlockSpec((1, D), lambda i: (0, 0)),
                  pl.BlockSpec((1, D), lambda i: (0, 0))] + extra_specs,
        out_specs=pl.BlockSpec((rb, D), lambda i: (i, 0)),
        out_shape=jax.ShapeDtypeStruct((n, D), jnp.float32),
    )(h_pad, parts, w1, b1, w2, b2, g, bt, m, v, *extra_in)


def _sc_aggregate(h_pad, packed2, ea_pad):
    epw = ea_pad.shape[0] // NW
    nblk = epw // BLK
    epw_pad = packed2.shape[1]
    rps = NPAD // NSUB
    mesh = plsc.VectorSubcoreMesh(core_axis_name="c", subcore_axis_name="s")
    offs = tuple(sorted(set(list(range(0, BLK - 15, 16)) + [BLK - 16])))

    @functools.partial(
        pl.kernel,
        out_type=jax.ShapeDtypeStruct((NCORES, NPAD, D), jnp.float32),
        mesh=mesh,
        scratch_types=[
            pltpu.VMEM((epw_pad,), jnp.int32),
            pltpu.VMEM((2, BLK), jnp.int32),
            pltpu.VMEM((2, BLK), jnp.int32),
            pltpu.VMEM((2, BLK, D), jnp.float32),
            pltpu.VMEM((2, BLK, D), jnp.float32),
            pltpu.VMEM((2, BLK, D), jnp.float32),
            pltpu.VMEM_SHARED((NPAD, D), jnp.float32),
            pltpu.SemaphoreType.DMA,
            pltpu.SemaphoreType.DMA,
            pltpu.SemaphoreType.DMA,
            pltpu.SemaphoreType.DMA,
            pltpu.SemaphoreType.DMA,
            pltpu.SemaphoreType.DMA,
            pltpu.SemaphoreType.DMA,
            pltpu.SemaphoreType.DMA,
        ],
    )
    def k(h_hbm, pk_hbm, ea_hbm, out_hbm,
          pk_all, sidx, didx, gat, eab, msg, agg_sp,
          isem, zsem, gsem0, gsem1, esem0, esem1, ssem0, ssem1):
        core = lax.axis_index("c")
        sub = lax.axis_index("s")
        wid = sub * NCORES + core
        ebase = wid * epw
        gsem = (gsem0, gsem1)
        esem = (esem0, esem1)
        ssem = (ssem0, ssem1)

        pltpu.async_copy(pk_hbm.at[wid], pk_all, isem)

        for p in range(2):
            @pl.loop(0, BLK)
            def _(r):
                for j in range(D // 16):
                    msg.at[p, r, pl.ds(j * 16, 16)][...] = jnp.zeros(
                        (16,), jnp.float32)

        @pl.loop(0, rps // BLK)
        def _(t):
            pltpu.async_copy(
                msg.at[0], agg_sp.at[pl.ds(sub * rps + t * BLK, BLK)], zsem)
        pltpu.make_async_copy(pk_hbm.at[wid], pk_all, isem).wait()

        @pl.loop(0, rps // BLK)
        def _(t):
            pltpu.make_async_copy(
                msg.at[0], agg_sp.at[pl.ds(sub * rps + t * BLK, BLK)],
                zsem).wait()

        def unpack_src(b, p):
            for off in offs:
                sidx.at[p, pl.ds(off, 16)][...] = jnp.bitwise_and(
                    pk_all.at[pl.ds(b * BLK + off, 16)][...], 16383)

        def unpack_dst(b, p):
            for off in offs:
                didx.at[p, pl.ds(off, 16)][...] = jax.lax.shift_right_logical(
                    pk_all.at[pl.ds(b * BLK + off, 16)][...], 14)

        def fetch(b, p):
            pltpu.async_copy(ea_hbm.at[pl.ds(ebase + b * BLK, BLK)],
                             eab.at[p], esem[p])
            pltpu.async_copy(h_hbm.at[sidx.at[p]], gat.at[p], gsem[p])

        def wait_fetch(b, p):
            pltpu.make_async_copy(ea_hbm.at[pl.ds(ebase + b * BLK, BLK)],
                                  eab.at[p], esem[p]).wait()
            pltpu.make_async_copy(h_hbm.at[sidx.at[p]], gat.at[p],
                                  gsem[p]).wait()

        def wait_scat(b, p):
            pltpu.make_async_copy(msg.at[p], agg_sp.at[didx.at[p]],
                                  ssem[p]).wait()

        def compute(p):
            @pl.loop(0, BLK, unroll=4)
            def _(r):
                for j in range(D // 16 - 1):
                    sl = pl.ds(j * 16, 16)
                    msg.at[p, r, sl][...] = jnp.maximum(
                        gat.at[p, r, sl][...] + eab.at[p, r, sl][...], 0.0)

        def scat(p):
            pltpu.async_copy(msg.at[p], agg_sp.at[didx.at[p]],
                             ssem[p], add=True)

        unpack_src(0, 0)
        unpack_src(1, 1)
        plsc.subcore_barrier()
        fetch(0, 0)
        fetch(1, 1)

        @pl.loop(0, nblk // 2)
        def _(t):
            for p in range(2):
                b = 2 * t + p
                wait_fetch(b, p)

                @pl.when(t > 0)
                def _():
                    wait_scat(b - 2, p)

                unpack_dst(b, p)
                compute(p)
                scat(p)

                @pl.when(b + 2 < nblk)
                def _():
                    unpack_src(b + 2, p)
                    fetch(b + 2, p)

        if nblk % 2:
            b = nblk - 1
            wait_fetch(b, 0)
            wait_scat(b - 2, 0)
            unpack_dst(b, 0)
            compute(0)
            scat(0)
            wait_scat(b - 1, 1)
            wait_scat(b, 0)
        else:
            wait_scat(nblk - 2, 0)
            wait_scat(nblk - 1, 1)
        plsc.subcore_barrier()

        @pl.loop(0, rps // ZROWS)
        def _(t):
            r0 = sub * rps + t * ZROWS
            pltpu.async_copy(agg_sp.at[pl.ds(r0, ZROWS)],
                             out_hbm.at[core, pl.ds(r0, ZROWS)], isem)

        @pl.loop(0, rps // ZROWS)
        def _(t):
            r0 = sub * rps + t * ZROWS
            pltpu.make_async_copy(agg_sp.at[pl.ds(r0, ZROWS)],
                                  out_hbm.at[core, pl.ds(r0, ZROWS)],
                                  isem).wait()

    return k(h_pad, packed2, ea_pad)


def kernel(x, edge_index, edge_attr, Wn, bn_, We, be, conv_W1, conv_b1,
           conv_W2, conv_b2, bn_gamma, bn_beta, bn_mean, bn_var,
           mW1, mb1, mW2, mb2, mW3, mb3):
    h = _proj(x, _pad2(Wn, Wn.shape[0], D), _pad1(bn_, D), 2000)
    ea = _proj(edge_attr, _pad2(We, edge_attr.shape[1], D), _pad1(be, D), 4000)
    e = edge_index.shape[1]
    epw = e // NW
    epw_pad = -(-epw // D) * D
    packed = jnp.pad((edge_index[0] + edge_index[1] * 16384).reshape(NW, epw),
                     ((0, 0), (0, epw_pad - epw)))

    head = (_pad2(mW1, D, 64), _pad1(mb1, 64),
            _pad2(mW2, 64, 32), _pad1(mb2, 32),
            _pad2(mW3, 32, D), _pad1(mb3, D))

    out_pad = None
    for i in range(conv_W1.shape[0]):
        parts = _sc_aggregate(h, packed, ea)
        args = (h, parts,
                _pad2(conv_W1[i], D, D), _pad1(conv_b1[i], D),
                _pad2(conv_W2[i], D, D), _pad1(conv_b2[i], D),
                _pad1(bn_gamma[i], D), _pad1(bn_beta[i], D),
                _pad1(bn_mean[i], D), _pad1(bn_var[i], D, fill=1.0))
        if i + 1 < conv_W1.shape[0]:
            h = _layer_update(*args)
        else:
            out_pad = _layer_update(*args, head=head)
    return out_pad[:, :mW3.shape[1]]


# speedup vs baseline: 1.7206x; 1.7061x over previous
import functools

import jax
import jax.numpy as jnp
from jax import lax
from jax.experimental import pallas as pl
from jax.experimental.pallas import tpu as pltpu
from jax.experimental.pallas import tpu_sc as plsc

D = 128
NCORES = 2
NSUB = 16
NW = NCORES * NSUB
BLK = 40
NPAD = 10240
ZROWS = 160

def _mm(a, b):
    return jax.lax.dot_general(a, b, (((1,), (0,)), ((), ())),
                               preferred_element_type=jnp.float32)


def _pad2(w, r, c):
    return jnp.pad(w, ((0, r - w.shape[0]), (0, c - w.shape[1])))


def _pad1(b, n, fill=0.0):
    return jnp.pad(b, (0, n - b.shape[0]), constant_values=fill).reshape(1, n)


def _proj(x, w_p, b_p, rb):
    n, k = x.shape

    def body(x_ref, w_ref, b_ref, o_ref):
        o_ref[...] = _mm(x_ref[...], w_ref[...]) + b_ref[...]

    return pl.pallas_call(
        body,
        grid=(n // rb,),
        in_specs=[pl.BlockSpec((rb, k), lambda i: (i, 0)),
                  pl.BlockSpec((k, D), lambda i: (0, 0)),
                  pl.BlockSpec((1, D), lambda i: (0, 0))],
        out_specs=pl.BlockSpec((rb, D), lambda i: (i, 0)),
        out_shape=jax.ShapeDtypeStruct((n, D), jnp.float32),
    )(x, w_p, b_p)


def _layer_update(h_pad, parts, w1, b1, w2, b2, g, bt, m, v, head=None):
    n = h_pad.shape[0]
    rb = 2000

    def update(h, agg, w1r, b1r, w2r, b2r, gr, btr, mr, vr):
        z = h + agg
        z = _mm(jnp.maximum(_mm(z, w1r) + b1r, 0.0), w2r) + b2r
        z = (z - mr) * jax.lax.rsqrt(vr + 1e-5) * gr + btr
        return (h + jnp.maximum(z, 0.0)) * 0.5

    if head is None:
        def body(h_ref, p_ref, w1r, b1r, w2r, b2r, gr, btr, mr, vr,
                 o_ref):
            o_ref[...] = update(h_ref[...], p_ref[0] + p_ref[1],
                                w1r[...], b1r[...], w2r[...], b2r[...],
                                gr[...], btr[...], mr[...], vr[...])
        extra_in, extra_specs = [], []
    else:
        mw1, mb1, mw2, mb2, mw3, mb3 = head

        def body(h_ref, p_ref, w1r, b1r, w2r, b2r, gr, btr, mr, vr,
                 mw1r, mb1r, mw2r, mb2r, mw3r, mb3r, o_ref):
            hn = update(h_ref[...], p_ref[0] + p_ref[1],
                        w1r[...], b1r[...], w2r[...], b2r[...],
                        gr[...], btr[...], mr[...], vr[...])
            o1 = jnp.maximum(_mm(hn, mw1r[...]) + mb1r[...], 0.0)
            o2 = jnp.maximum(_mm(o1, mw2r[...]) + mb2r[...], 0.0)
            o_ref[...] = _mm(o2, mw3r[...]) + mb3r[...]
        extra_in = [mw1, mb1, mw2, mb2, mw3, mb3]
        extra_specs = [
            pl.BlockSpec((D, 64), lambda i: (0, 0)),
            pl.BlockSpec((1, 64), lambda i: (0, 0)),
            pl.BlockSpec((64, 32), lambda i: (0, 0)),
            pl.BlockSpec((1, 32), lambda i: (0, 0)),
            pl.BlockSpec((32, D), lambda i: (0, 0)),
            pl.BlockSpec((1, D), lambda i: (0, 0)),
        ]

    return pl.pallas_call(
        body,
        grid=(n // rb,),
        in_specs=[pl.BlockSpec((rb, D), lambda i: (i, 0)),
                  pl.BlockSpec((NCORES, rb, D), lambda i: (0, i, 0)),
                  pl.BlockSpec((D, D), lambda i: (0, 0)),
                  pl.BlockSpec((1, D), lambda i: (0, 0)),
                  pl.BlockSpec((D, D), lambda i: (0, 0)),
                  pl.BlockSpec((1, D), lambda i: (0, 0)),
                  pl.BlockSpec((1, D), lambda i: (0, 0)),
                  pl.BlockSpec((1, D), lambda i: (0, 0)),
                  pl.BlockSpec((1, D), lambda i: (0, 0)),
                  pl.BlockSpec((1, D), lambda i: (0, 0))] + extra_specs,
        out_specs=pl.BlockSpec((rb, D), lambda i: (i, 0)),
        out_shape=jax.ShapeDtypeStruct((n, D), jnp.float32),
    )(h_pad, parts, w1, b1, w2, b2, g, bt, m, v, *extra_in)


def _sc_aggregate(h_pad, packed2, ea_pad):
    epw = ea_pad.shape[0] // NW
    nblk = epw // BLK
    epw_pad = packed2.shape[1]
    rps = NPAD // NSUB
    mesh = plsc.VectorSubcoreMesh(core_axis_name="c", subcore_axis_name="s")
    offs = tuple(sorted(set(list(range(0, BLK - 15, 16)) + [BLK - 16])))

    @functools.partial(
        pl.kernel,
        out_type=jax.ShapeDtypeStruct((NCORES, NPAD, D), jnp.float32),
        mesh=mesh,
        scratch_types=[
            pltpu.VMEM((epw_pad,), jnp.int32),
            pltpu.VMEM((2, BLK), jnp.int32),
            pltpu.VMEM((2, BLK), jnp.int32),
            pltpu.VMEM((2, BLK, D), jnp.float32),
            pltpu.VMEM((2, BLK, D), jnp.float32),
            pltpu.VMEM((2, BLK, D), jnp.float32),
            pltpu.VMEM_SHARED((NPAD, D), jnp.float32),
            pltpu.SemaphoreType.DMA,
            pltpu.SemaphoreType.DMA,
            pltpu.SemaphoreType.DMA,
            pltpu.SemaphoreType.DMA,
            pltpu.SemaphoreType.DMA,
            pltpu.SemaphoreType.DMA,
            pltpu.SemaphoreType.DMA,
            pltpu.SemaphoreType.DMA,
        ],
    )
    def k(h_hbm, pk_hbm, ea_hbm, out_hbm,
          pk_all, sidx, didx, gat, eab, msg, agg_sp,
          isem, zsem, gsem0, gsem1, esem0, esem1, ssem0, ssem1):
        core = lax.axis_index("c")
        sub = lax.axis_index("s")
        wid = sub * NCORES + core
        ebase = wid * epw
        gsem = (gsem0, gsem1)
        esem = (esem0, esem1)
        ssem = (ssem0, ssem1)

        pltpu.async_copy(pk_hbm.at[wid], pk_all, isem)

        for p in range(2):
            @pl.loop(0, BLK)
            def _(r):
                for j in range(D // 16):
                    msg.at[p, r, pl.ds(j * 16, 16)][...] = jnp.zeros(
                        (16,), jnp.float32)

        @pl.loop(0, rps // BLK)
        def _(t):
            pltpu.async_copy(
                msg.at[0], agg_sp.at[pl.ds(sub * rps + t * BLK, BLK)], zsem)
        pltpu.make_async_copy(pk_hbm.at[wid], pk_all, isem).wait()

        @pl.loop(0, rps // BLK)
        def _(t):
            pltpu.make_async_copy(
                msg.at[0], agg_sp.at[pl.ds(sub * rps + t * BLK, BLK)],
                zsem).wait()

        def unpack_src(b, p):
            for off in offs:
                sidx.at[p, pl.ds(off, 16)][...] = jnp.bitwise_and(
                    pk_all.at[pl.ds(b * BLK + off, 16)][...], 16383)

        def unpack_dst(b, p):
            for off in offs:
                didx.at[p, pl.ds(off, 16)][...] = jax.lax.shift_right_logical(
                    pk_all.at[pl.ds(b * BLK + off, 16)][...], 14)

        def fetch(b, p):
            pltpu.async_copy(ea_hbm.at[pl.ds(ebase + b * BLK, BLK)],
                             eab.at[p], esem[p])
            pltpu.async_copy(h_hbm.at[sidx.at[p]], gat.at[p], gsem[p])

        def wait_fetch(b, p):
            pltpu.make_async_copy(ea_hbm.at[pl.ds(ebase + b * BLK, BLK)],
                                  eab.at[p], esem[p]).wait()
            pltpu.make_async_copy(h_hbm.at[sidx.at[p]], gat.at[p],
                                  gsem[p]).wait()

        def wait_scat(b, p):
            pltpu.make_async_copy(msg.at[p], agg_sp.at[didx.at[p]],
                                  ssem[p]).wait()

        def compute(p):
            @plsc.parallel_loop(0, BLK)
            def _(r):
                for j in range(D // 16 - 1):
                    sl = pl.ds(j * 16, 16)
                    msg.at[p, r, sl][...] = jnp.maximum(
                        gat.at[p, r, sl][...] + eab.at[p, r, sl][...], 0.0)

        def scat(p):
            pltpu.async_copy(msg.at[p], agg_sp.at[didx.at[p]],
                             ssem[p], add=True)

        unpack_src(0, 0)
        unpack_src(1, 1)
        plsc.subcore_barrier()
        fetch(0, 0)
        fetch(1, 1)

        @pl.loop(0, nblk // 2)
        def _(t):
            for p in range(2):
                b = 2 * t + p
                wait_fetch(b, p)

                @pl.when(t > 0)
                def _():
                    wait_scat(b - 2, p)

                unpack_dst(b, p)
                compute(p)
                scat(p)

                @pl.when(b + 2 < nblk)
                def _():
                    unpack_src(b + 2, p)
                    fetch(b + 2, p)

        if nblk % 2:
            b = nblk - 1
            wait_fetch(b, 0)
            wait_scat(b - 2, 0)
            unpack_dst(b, 0)
            compute(0)
            scat(0)
            wait_scat(b - 1, 1)
            wait_scat(b, 0)
        else:
            wait_scat(nblk - 2, 0)
            wait_scat(nblk - 1, 1)
        plsc.subcore_barrier()

        @pl.loop(0, rps // ZROWS)
        def _(t):
            r0 = sub * rps + t * ZROWS
            pltpu.async_copy(agg_sp.at[pl.ds(r0, ZROWS)],
                             out_hbm.at[core, pl.ds(r0, ZROWS)], isem)

        @pl.loop(0, rps // ZROWS)
        def _(t):
            r0 = sub * rps + t * ZROWS
            pltpu.make_async_copy(agg_sp.at[pl.ds(r0, ZROWS)],
                                  out_hbm.at[core, pl.ds(r0, ZROWS)],
                                  isem).wait()

    return k(h_pad, packed2, ea_pad)


def kernel(x, edge_index, edge_attr, Wn, bn_, We, be, conv_W1, conv_b1,
           conv_W2, conv_b2, bn_gamma, bn_beta, bn_mean, bn_var,
           mW1, mb1, mW2, mb2, mW3, mb3):
    h = _proj(x, _pad2(Wn, Wn.shape[0], D), _pad1(bn_, D), 2000)
    ea = _proj(edge_attr, _pad2(We, edge_attr.shape[1], D), _pad1(be, D), 4000)
    e = edge_index.shape[1]
    epw = e // NW
    epw_pad = -(-epw // D) * D
    packed = jnp.pad((edge_index[0] + edge_index[1] * 16384).reshape(NW, epw),
                     ((0, 0), (0, epw_pad - epw)))

    head = (_pad2(mW1, D, 64), _pad1(mb1, 64),
            _pad2(mW2, 64, 32), _pad1(mb2, 32),
            _pad2(mW3, 32, D), _pad1(mb3, D))

    out_pad = None
    for i in range(conv_W1.shape[0]):
        parts = _sc_aggregate(h, packed, ea)
        args = (h, parts,
                _pad2(conv_W1[i], D, D), _pad1(conv_b1[i], D),
                _pad2(conv_W2[i], D, D), _pad1(conv_b2[i], D),
                _pad1(bn_gamma[i], D), _pad1(bn_beta[i], D),
                _pad1(bn_mean[i], D), _pad1(bn_var[i], D, fill=1.0))
        if i + 1 < conv_W1.shape[0]:
            h = _layer_update(*args)
        else:
            out_pad = _layer_update(*args, head=head)
    return out_pad[:, :mW3.shape[1]]
